# baseline (speedup 1.0000x reference)
import functools

import jax
import jax.numpy as jnp
from jax import lax
from jax.experimental import pallas as pl
from jax.experimental.pallas import tpu as pltpu
from jax.experimental.pallas import tpu_sc as plsc

EMB = 64
SCALE = 8.0

NC = 2
NS = 16
NW = NC * NS
LANES = 16

B = 16384
L = 200
NBI = 8
NB = 2
PER_W = B // NW
NIT = PER_W // NB


def _gather_scale(tok, table128):
    mesh = plsc.VectorSubcoreMesh(core_axis_name="c", subcore_axis_name="s")

    @functools.partial(
        pl.kernel,
        mesh=mesh,
        out_type=jax.ShapeDtypeStruct((B, L, EMB), jnp.float32),
        scratch_types=[
            pltpu.VMEM((NBI, L), jnp.int32),
            pltpu.VMEM((NB, L, 128), jnp.float32),
            pltpu.VMEM((NB, L, EMB), jnp.float32),
            pltpu.SemaphoreType.DMA,
        ],
    )
    def k(tok_hbm, tab_hbm, out_hbm, idx_v, rows_v, outc_v, sem):
        wid = lax.axis_index("s") * NC + lax.axis_index("c")
        base = wid * PER_W

        def body(i, carry):
            b0 = pl.multiple_of(base + i * NB, NB)

            @pl.when(i % (NBI // NB) == 0)
            def _fetch_idx():
                pltpu.sync_copy(
                    tok_hbm.at[pl.ds(pl.multiple_of(b0, NBI), NBI)], idx_v
                )

            ro = (i % (NBI // NB)) * NB
            for r in range(NB):
                pltpu.async_copy(
                    tab_hbm.at[idx_v.at[ro + r, pl.ds(0, 128)]],
                    rows_v.at[r, pl.ds(0, 128)],
                    sem,
                )
                pltpu.async_copy(
                    tab_hbm.at[idx_v.at[ro + r, pl.ds(128, 72)]],
                    rows_v.at[r, pl.ds(128, 72)],
                    sem,
                )
            for r in range(NB):
                pltpu.make_async_copy(
                    tab_hbm.at[idx_v.at[ro + r, pl.ds(0, 128)]],
                    rows_v.at[r, pl.ds(0, 128)],
                    sem,
                ).wait()
                pltpu.make_async_copy(
                    tab_hbm.at[idx_v.at[ro + r, pl.ds(128, 72)]],
                    rows_v.at[r, pl.ds(128, 72)],
                    sem,
                ).wait()

            def scale_l(l, c2):
                for r in range(NB):
                    for c in range(EMB // LANES):
                        src = (r, l, pl.ds(c * LANES, LANES))
                        outc_v[src] = rows_v[src] * SCALE
                return c2

            lax.fori_loop(0, L, scale_l, 0)
            pltpu.sync_copy(outc_v, out_hbm.at[pl.ds(b0, NB)])
            return carry

        lax.fori_loop(0, NIT, body, 0)

    return k(tok, table128)


def kernel(token_tensor, embedding_table):
    tab128 = jnp.pad(embedding_table, ((0, 0), (0, 128 - EMB)))
    return _gather_scale(token_tensor.astype(jnp.int32), tab128)


# speedup vs baseline: 1.4238x; 1.4238x over previous
import functools

import jax
import jax.numpy as jnp
from jax import lax
from jax.experimental import pallas as pl
from jax.experimental.pallas import tpu as pltpu
from jax.experimental.pallas import tpu_sc as plsc

EMB = 64
SCALE = 8.0

NC = 2
NS = 16
NW = NC * NS
LANES = 16

B = 16384
L = 200
LPAD = 256
NB = 8
PER_W = B // NW
NIT = PER_W // NB


def _gather_scale(tok, table):
    mesh = plsc.VectorSubcoreMesh(core_axis_name="c", subcore_axis_name="s")

    @functools.partial(
        pl.kernel,
        mesh=mesh,
        out_type=jax.ShapeDtypeStruct((B, L, 128), jnp.float32),
        scratch_types=[
            pltpu.VMEM((NB, LPAD), jnp.int32),
            pltpu.VMEM((NB, L, EMB), jnp.float32),
            pltpu.SemaphoreType.DMA,
        ],
        compiler_params=pltpu.CompilerParams(use_tc_tiling_on_sc=False),
    )
    def k(tok_hbm, tab_hbm, out_hbm, idx_v, rows_v, sem):
        wid = lax.axis_index("s") * NC + lax.axis_index("c")
        base = wid * PER_W

        def body(i, carry):
            b0 = pl.multiple_of(base + i * NB, NB)
            pltpu.sync_copy(tok_hbm.at[pl.ds(b0, NB)], idx_v)
            for r in range(NB):
                pltpu.async_copy(
                    tab_hbm.at[idx_v.at[r, pl.ds(0, 128)]],
                    rows_v.at[r, pl.ds(0, 128)],
                    sem,
                )
                pltpu.async_copy(
                    tab_hbm.at[idx_v.at[r, pl.ds(128, 72)]],
                    rows_v.at[r, pl.ds(128, 72)],
                    sem,
                )
            for r in range(NB):
                pltpu.make_async_copy(
                    tab_hbm.at[idx_v.at[r, pl.ds(0, 128)]],
                    rows_v.at[r, pl.ds(0, 128)],
                    sem,
                ).wait()
                pltpu.make_async_copy(
                    tab_hbm.at[idx_v.at[r, pl.ds(128, 72)]],
                    rows_v.at[r, pl.ds(128, 72)],
                    sem,
                ).wait()

            pltpu.sync_copy(
                rows_v, out_hbm.at[pl.ds(b0, NB), :, pl.ds(0, EMB)]
            )
            return carry

        lax.fori_loop(0, NIT, body, 0)

    return k(tok, table)


def kernel(token_tensor, embedding_table):
    tok = jnp.pad(token_tensor.astype(jnp.int32), ((0, 0), (0, LPAD - L)))
    out = _gather_scale(tok, embedding_table * SCALE)
    return out[:, :, :EMB]


# speedup vs baseline: 1.6906x; 1.1874x over previous
import functools

import jax
import jax.numpy as jnp
from jax import lax
from jax.experimental import pallas as pl
from jax.experimental.pallas import tpu as pltpu
from jax.experimental.pallas import tpu_sc as plsc

EMB = 64
SCALE = 8.0

NC = 2
NS = 16
NW = NC * NS
LANES = 16

B = 16384
L = 200
LPAD = 256
NB = 2
NSUB = 4
NBI = NB * NSUB
PER_W = B // NW
NBLK = PER_W // NBI


def _gather_scale(tok, table):
    mesh = plsc.VectorSubcoreMesh(core_axis_name="c", subcore_axis_name="s")

    @functools.partial(
        pl.kernel,
        mesh=mesh,
        out_type=jax.ShapeDtypeStruct((B, L, 128), jnp.float32),
        scratch_types=[
            pltpu.VMEM((3, NBI, LPAD), jnp.int32),
            pltpu.VMEM((NSUB, NB, L, EMB), jnp.float32),
            pltpu.SemaphoreType.DMA,
            pltpu.SemaphoreType.DMA,
            pltpu.SemaphoreType.DMA,
            pltpu.SemaphoreType.DMA,
            pltpu.SemaphoreType.DMA,
            pltpu.SemaphoreType.DMA,
            pltpu.SemaphoreType.DMA,
            pltpu.SemaphoreType.DMA,
            pltpu.SemaphoreType.DMA,
        ],
        compiler_params=pltpu.CompilerParams(use_tc_tiling_on_sc=False),
    )
    def k(tok_hbm, tab_hbm, out_hbm, idx_v, rows_v,
          si, sg0, sg1, sg2, sg3, ss0, ss1, ss2, ss3):
        sg = [sg0, sg1, sg2, sg3]
        ss = [ss0, ss1, ss2, ss3]
        wid = lax.axis_index("s") * NC + lax.axis_index("c")
        base = wid * PER_W

        def stream_pairs(slot, ro, buf):
            for r in range(NB):
                yield (tab_hbm.at[idx_v.at[slot, ro + r, pl.ds(0, 128)]],
                       rows_v.at[buf, r, pl.ds(0, 128)])
                yield (tab_hbm.at[idx_v.at[slot, ro + r, pl.ds(128, 72)]],
                       rows_v.at[buf, r, pl.ds(128, 72)])

        def fire(slot, ro, buf):
            for src, dst in stream_pairs(slot, ro, buf):
                pltpu.async_copy(src, dst, sg[buf])

        def drain(buf):
            for src, dst in stream_pairs(0, 0, buf):
                pltpu.make_async_copy(src, dst, sg[buf]).wait()

        def scale(buf):
            def sbody(l, c2):
                for r in range(NB):
                    for c in range(EMB // LANES):
                        sl = (buf, r, l, pl.ds(c * LANES, LANES))
                        rows_v[sl] = rows_v[sl] * SCALE
                return c2

            lax.fori_loop(0, L, sbody, 0)

        def store(buf, row0):
            pltpu.async_copy(
                rows_v.at[buf],
                out_hbm.at[pl.ds(row0, NB), :, pl.ds(0, EMB)],
                ss[buf],
            )

        def wait_store(buf):
            pltpu.make_async_copy(
                rows_v.at[buf],
                out_hbm.at[pl.ds(base, NB), :, pl.ds(0, EMB)],
                ss[buf],
            ).wait()

        def wait_idx():
            pltpu.make_async_copy(
                tok_hbm.at[pl.ds(base, NBI)], idx_v.at[0], si
            ).wait()

        pltpu.sync_copy(tok_hbm.at[pl.ds(base, NBI)], idx_v.at[0])
        pltpu.async_copy(tok_hbm.at[pl.ds(base + NBI, NBI)], idx_v.at[1], si)
        fire(0, 0, 0)
        fire(0, NB, 1)

        def gbody(g, carry):
            slot_g = lax.rem(g, 3)
            slot_g1 = lax.rem(g + 1, 3)

            @pl.when(g + 2 < NBLK)
            def _prefetch():
                pltpu.async_copy(
                    tok_hbm.at[pl.ds(base + (g + 2) * NBI, NBI)],
                    idx_v.at[lax.rem(g + 2, 3)],
                    si,
                )

            for j in range(NSUB):
                drain(j)
                scale(j)
                tbuf = (j + 2) % NSUB
                if j < 2:
                    @pl.when(g >= 1)
                    def _ws():
                        wait_store(tbuf)

                    fire(slot_g, (j + 2) * NB, tbuf)
                else:
                    @pl.when(g < NBLK - 1)
                    def _fire_next():
                        if j == 2:
                            wait_idx()
                        wait_store(tbuf)
                        fire(slot_g1, (j - 2) * NB, tbuf)

                store(j, base + (NSUB * g + j) * NB)
            return carry

        lax.fori_loop(0, NBLK, gbody, 0)
        for t in range(NSUB):
            wait_store(t)

    return k(tok, table)


def kernel(token_tensor, embedding_table):
    tok = jnp.pad(token_tensor.astype(jnp.int32), ((0, 0), (0, LPAD - L)))
    out = _gather_scale(tok, embedding_table)
    return out[:, :, :EMB]


# speedup vs baseline: 1.6912x; 1.0003x over previous
import functools

import jax
import jax.numpy as jnp
from jax import lax
from jax.experimental import pallas as pl
from jax.experimental.pallas import tpu as pltpu
from jax.experimental.pallas import tpu_sc as plsc

EMB = 64
SCALE = 8.0

NC = 2
NS = 16
NW = NC * NS
LANES = 16

B = 16384
L = 200
LPAD = 256
NB = 2
NSUB = 4
NBI = NB * NSUB
PER_W = B // NW
NBLK = PER_W // NBI


def _gather_scale(tok, table):
    mesh = plsc.VectorSubcoreMesh(core_axis_name="c", subcore_axis_name="s")

    @functools.partial(
        pl.kernel,
        mesh=mesh,
        out_type=jax.ShapeDtypeStruct((B, L, 128), jnp.float32),
        scratch_types=[
            pltpu.VMEM((3, NBI, LPAD), jnp.int32),
            pltpu.VMEM((NSUB, NB, L, EMB), jnp.float32),
            pltpu.SemaphoreType.DMA,
            pltpu.SemaphoreType.DMA,
            pltpu.SemaphoreType.DMA,
            pltpu.SemaphoreType.DMA,
            pltpu.SemaphoreType.DMA,
            pltpu.SemaphoreType.DMA,
            pltpu.SemaphoreType.DMA,
            pltpu.SemaphoreType.DMA,
            pltpu.SemaphoreType.DMA,
        ],
        compiler_params=pltpu.CompilerParams(use_tc_tiling_on_sc=False),
    )
    def k(tab_hbm, tok_hbm, out_hbm, idx_v, rows_v,
          si, sg0, sg1, sg2, sg3, ss0, ss1, ss2, ss3):
        sg = [sg0, sg1, sg2, sg3]
        ss = [ss0, ss1, ss2, ss3]
        wid = lax.axis_index("s") * NC + lax.axis_index("c")
        base = wid * PER_W

        def stream_pairs(slot, ro, buf):
            for r in range(NB):
                yield (tab_hbm.at[idx_v.at[slot, ro + r, pl.ds(0, 128)]],
                       rows_v.at[buf, r, pl.ds(0, 128)])
                yield (tab_hbm.at[idx_v.at[slot, ro + r, pl.ds(128, 72)]],
                       rows_v.at[buf, r, pl.ds(128, 72)])

        def fire(slot, ro, buf):
            for src, dst in stream_pairs(slot, ro, buf):
                pltpu.async_copy(src, dst, sg[buf])

        def drain(buf):
            for src, dst in stream_pairs(0, 0, buf):
                pltpu.make_async_copy(src, dst, sg[buf]).wait()

        def scale(buf):
            def sbody(l, c2):
                for r in range(NB):
                    for c in range(EMB // LANES):
                        sl = (buf, r, l, pl.ds(c * LANES, LANES))
                        rows_v[sl] = rows_v[sl] * SCALE
                return c2

            lax.fori_loop(0, L, sbody, 0)

        def store(buf, row0):
            pltpu.async_copy(
                rows_v.at[buf],
                out_hbm.at[pl.ds(row0, NB), :, pl.ds(0, EMB)],
                ss[buf],
            )

        def wait_store(buf):
            pltpu.make_async_copy(
                rows_v.at[buf],
                out_hbm.at[pl.ds(base, NB), :, pl.ds(0, EMB)],
                ss[buf],
            ).wait()

        def wait_idx():
            pltpu.make_async_copy(
                tok_hbm.at[pl.ds(base, NBI)], idx_v.at[0], si
            ).wait()

        pltpu.sync_copy(tok_hbm.at[pl.ds(base, NBI)], idx_v.at[0])
        pltpu.async_copy(tok_hbm.at[pl.ds(base + NBI, NBI)], idx_v.at[1], si)
        fire(0, 0, 0)
        fire(0, NB, 1)

        def gbody(g, carry):
            slot_g = lax.rem(g, 3)
            slot_g1 = lax.rem(g + 1, 3)

            @pl.when(g + 2 < NBLK)
            def _prefetch():
                pltpu.async_copy(
                    tok_hbm.at[pl.ds(base + (g + 2) * NBI, NBI)],
                    idx_v.at[lax.rem(g + 2, 3)],
                    si,
                )

            for j in range(NSUB):
                drain(j)
                scale(j)
                tbuf = (j + 2) % NSUB
                if j < 2:
                    @pl.when(g >= 1)
                    def _ws():
                        wait_store(tbuf)

                    fire(slot_g, (j + 2) * NB, tbuf)
                else:
                    @pl.when(g < NBLK - 1)
                    def _fire_next():
                        if j == 2:
                            wait_idx()
                        wait_store(tbuf)
                        fire(slot_g1, (j - 2) * NB, tbuf)

                store(j, base + (NSUB * g + j) * NB)
            return carry

        lax.fori_loop(0, NBLK, gbody, 0)
        for t in range(NSUB):
            wait_store(t)

    return k(table, tok)


def kernel(token_tensor, embedding_table):
    tok = jnp.pad(token_tensor.astype(jnp.int32), ((0, 0), (0, LPAD - L)))
    out = _gather_scale(tok, embedding_table)
    return out[:, :, :EMB]


# speedup vs baseline: 1.6921x; 1.0005x over previous
import functools

import jax
import jax.numpy as jnp
from jax import lax
from jax.experimental import pallas as pl
from jax.experimental.pallas import tpu as pltpu
from jax.experimental.pallas import tpu_sc as plsc

EMB = 64
SCALE = 8.0

NC = 2
NS = 16
NW = NC * NS
LANES = 16

B = 16384
L = 200
LPAD = 256
NB = 2
NSUB = 4
NBI = NB * NSUB
PER_W = B // NW
NBLK = PER_W // NBI


def _gather_scale(tok, table):
    mesh = plsc.VectorSubcoreMesh(core_axis_name="c", subcore_axis_name="s")

    @functools.partial(
        pl.kernel,
        mesh=mesh,
        out_type=jax.ShapeDtypeStruct((B, L, 128), jnp.float32),
        scratch_types=[
            pltpu.VMEM((3, NBI, LPAD), jnp.int32),
            pltpu.VMEM((NSUB, NB, L, EMB), jnp.float32),
            pltpu.SemaphoreType.DMA,
            pltpu.SemaphoreType.DMA,
            pltpu.SemaphoreType.DMA,
            pltpu.SemaphoreType.DMA,
            pltpu.SemaphoreType.DMA,
            pltpu.SemaphoreType.DMA,
            pltpu.SemaphoreType.DMA,
            pltpu.SemaphoreType.DMA,
            pltpu.SemaphoreType.DMA,
        ],
        compiler_params=pltpu.CompilerParams(use_tc_tiling_on_sc=False),
    )
    def k(tab_hbm, tok_hbm, out_hbm, idx_v, rows_v,
          si, sg0, sg1, sg2, sg3, ss0, ss1, ss2, ss3):
        sg = [sg0, sg1, sg2, sg3]
        ss = [ss0, ss1, ss2, ss3]
        wid = lax.axis_index("s") * NC + lax.axis_index("c")
        base = wid * PER_W

        def stream_pairs(slot, ro, buf):
            for r in range(NB):
                yield (tab_hbm.at[idx_v.at[slot, ro + r, pl.ds(0, 128)]],
                       rows_v.at[buf, r, pl.ds(0, 128)])
                yield (tab_hbm.at[idx_v.at[slot, ro + r, pl.ds(128, 72)]],
                       rows_v.at[buf, r, pl.ds(128, 72)])

        def fire(slot, ro, buf):
            for src, dst in stream_pairs(slot, ro, buf):
                pltpu.async_copy(src, dst, sg[buf])

        def drain(buf):
            for src, dst in stream_pairs(0, 0, buf):
                pltpu.make_async_copy(src, dst, sg[buf]).wait()

        def scale(buf):
            def sbody(l, c2):
                for r in range(NB):
                    for c in range(EMB // LANES):
                        sl = (buf, r, l, pl.ds(c * LANES, LANES))
                        rows_v[sl] = rows_v[sl] * SCALE
                return c2

            lax.fori_loop(0, L, sbody, 0, unroll=4)

        def store(buf, row0):
            pltpu.async_copy(
                rows_v.at[buf],
                out_hbm.at[pl.ds(row0, NB), :, pl.ds(0, EMB)],
                ss[buf],
            )

        def wait_store(buf):
            pltpu.make_async_copy(
                rows_v.at[buf],
                out_hbm.at[pl.ds(base, NB), :, pl.ds(0, EMB)],
                ss[buf],
            ).wait()

        def wait_idx():
            pltpu.make_async_copy(
                tok_hbm.at[pl.ds(base, NBI)], idx_v.at[0], si
            ).wait()

        pltpu.sync_copy(tok_hbm.at[pl.ds(base, NBI)], idx_v.at[0])
        pltpu.async_copy(tok_hbm.at[pl.ds(base + NBI, NBI)], idx_v.at[1], si)
        fire(0, 0, 0)
        fire(0, NB, 1)

        def gbody(g, carry):
            slot_g = lax.rem(g, 3)
            slot_g1 = lax.rem(g + 1, 3)

            @pl.when(g + 2 < NBLK)
            def _prefetch():
                pltpu.async_copy(
                    tok_hbm.at[pl.ds(base + (g + 2) * NBI, NBI)],
                    idx_v.at[lax.rem(g + 2, 3)],
                    si,
                )

            for j in range(NSUB):
                drain(j)
                scale(j)
                tbuf = (j + 2) % NSUB
                if j < 2:
                    @pl.when(g >= 1)
                    def _ws():
                        wait_store(tbuf)

                    fire(slot_g, (j + 2) * NB, tbuf)
                else:
                    @pl.when(g < NBLK - 1)
                    def _fire_next():
                        if j == 2:
                            wait_idx()
                        wait_store(tbuf)
                        fire(slot_g1, (j - 2) * NB, tbuf)

                store(j, base + (NSUB * g + j) * NB)
            return carry

        lax.fori_loop(0, NBLK, gbody, 0)
        for t in range(NSUB):
            wait_store(t)

    return k(table, tok)


def kernel(token_tensor, embedding_table):
    tok = jnp.pad(token_tensor.astype(jnp.int32), ((0, 0), (0, LPAD - L)))
    out = _gather_scale(tok, embedding_table)
    return out[:, :, :EMB]
